# baseline (speedup 1.0000x reference)
import functools

import jax
import jax.numpy as jnp
from jax import lax
from jax.experimental import pallas as pl
from jax.experimental.pallas import tpu as pltpu
from jax.experimental.pallas import tpu_sc as plsc

N = 10000
NP = 10240
E0 = 160000
EP = 172032
NW = 32
EPW = EP // NW
C1 = 256
C2 = 32
NCH = EPW // C2
HP = 512
D = 128
HEADS = 4
PH = 128
G = 256
PR = 384

_mesh = plsc.VectorSubcoreMesh(core_axis_name="c", subcore_axis_name="s",
                               num_cores=2, num_subcores=16)


def _node_mm_body(h_ref, wt_ref, ct_ref, xw_ref, st_ref):
    h = h_ref[...]
    xw_ref[...] = jnp.dot(h, wt_ref[...], preferred_element_type=jnp.float32)
    st_ref[...] = lax.dot_general(ct_ref[...], h, (((1,), (1,)), ((), ())),
                                  preferred_element_type=jnp.float32)


def _node_matmul(h, wt, ct, blk=1024):
    return pl.pallas_call(
        _node_mm_body,
        grid=(NP // blk,),
        in_specs=[
            pl.BlockSpec((blk, D), lambda i: (i, 0)),
            pl.BlockSpec((D, HP), lambda i: (0, 0)),
            pl.BlockSpec((8, D), lambda i: (0, 0)),
        ],
        out_specs=[
            pl.BlockSpec((blk, HP), lambda i: (i, 0)),
            pl.BlockSpec((8, blk), lambda i: (0, i)),
        ],
        out_shape=[
            jax.ShapeDtypeStruct((NP, HP), jnp.float32),
            jax.ShapeDtypeStruct((8, NP), jnp.float32),
        ],
    )(h, wt, ct)


def _edge_mm_body(ea_ref, ce_ref, out_ref):
    out_ref[...] = jnp.dot(ea_ref[...], ce_ref[...],
                           preferred_element_type=jnp.float32)


def _edge_matmul(ea, ce, blk=8000):
    e = ea.shape[0]
    return pl.pallas_call(
        _edge_mm_body,
        grid=(e // blk,),
        in_specs=[
            pl.BlockSpec((blk, D), lambda i: (i, 0)),
            pl.BlockSpec((D, 8), lambda i: (0, 0)),
        ],
        out_specs=pl.BlockSpec((blk, 8), lambda i: (i, 0)),
        out_shape=jax.ShapeDtypeStruct((e, 8), jnp.float32),
    )(ea, ce)


def _ln_mm_body(agg_ref, h_ref, g_ref, b_ref, wt_ref, ct_ref,
                hn_ref, xw_ref, st_ref):
    agg = agg_ref[0] + agg_ref[1]
    mu = jnp.mean(agg, axis=-1, keepdims=True)
    var = jnp.mean((agg - mu) ** 2, axis=-1, keepdims=True)
    z = (agg - mu) * lax.rsqrt(var + 1e-5) * g_ref[...] + b_ref[...]
    z = jnp.where(z > 0, z, jnp.exp(z) - 1.0)
    hn = z + h_ref[...]
    hn_ref[...] = hn
    xw_ref[...] = jnp.dot(hn, wt_ref[...], preferred_element_type=jnp.float32)
    st_ref[...] = lax.dot_general(ct_ref[...], hn, (((1,), (1,)), ((), ())),
                                  preferred_element_type=jnp.float32)


def _ln_mm(agg2, h, g, b, wt, ct, blk=1024):
    return pl.pallas_call(
        _ln_mm_body,
        grid=(NP // blk,),
        in_specs=[
            pl.BlockSpec((2, blk, D), lambda i: (0, i, 0)),
            pl.BlockSpec((blk, D), lambda i: (i, 0)),
            pl.BlockSpec((1, D), lambda i: (0, 0)),
            pl.BlockSpec((1, D), lambda i: (0, 0)),
            pl.BlockSpec((D, HP), lambda i: (0, 0)),
            pl.BlockSpec((8, D), lambda i: (0, 0)),
        ],
        out_specs=[
            pl.BlockSpec((blk, D), lambda i: (i, 0)),
            pl.BlockSpec((blk, HP), lambda i: (i, 0)),
            pl.BlockSpec((8, blk), lambda i: (0, i)),
        ],
        out_shape=[
            jax.ShapeDtypeStruct((NP, D), jnp.float32),
            jax.ShapeDtypeStruct((NP, HP), jnp.float32),
            jax.ShapeDtypeStruct((8, NP), jnp.float32),
        ],
    )(agg2, h, g, b, wt, ct)


def _ln_body(agg_ref, h_ref, g_ref, b_ref, hn_ref):
    agg = agg_ref[0] + agg_ref[1]
    mu = jnp.mean(agg, axis=-1, keepdims=True)
    var = jnp.mean((agg - mu) ** 2, axis=-1, keepdims=True)
    z = (agg - mu) * lax.rsqrt(var + 1e-5) * g_ref[...] + b_ref[...]
    z = jnp.where(z > 0, z, jnp.exp(z) - 1.0)
    hn_ref[...] = z + h_ref[...]


def _ln_only(agg2, h, g, b, blk=1024):
    return pl.pallas_call(
        _ln_body,
        grid=(NP // blk,),
        in_specs=[
            pl.BlockSpec((2, blk, D), lambda i: (0, i, 0)),
            pl.BlockSpec((blk, D), lambda i: (i, 0)),
            pl.BlockSpec((1, D), lambda i: (0, 0)),
            pl.BlockSpec((1, D), lambda i: (0, 0)),
        ],
        out_specs=pl.BlockSpec((blk, D), lambda i: (i, 0)),
        out_shape=jax.ShapeDtypeStruct((NP, D), jnp.float32),
    )(agg2, h, g, b)


def _mlp_body(p_ref, c_ref, l1_ref, b1_ref, l2_ref, b2_ref, hh_ref, out_ref):
    p = p_ref[0:PR, :] + p_ref[PR:2 * PR, :]
    po = p[0:G, :]
    cnt = jnp.sum(c_ref[...], axis=0)[0:G]
    pooled = po / jnp.maximum(cnt, 1.0)[:, None]
    hh = jnp.dot(pooled, l1_ref[...], preferred_element_type=jnp.float32) + b1_ref[...]
    hh_ref[...] = hh
    out_ref[...] = jnp.dot(hh, l2_ref[...], preferred_element_type=jnp.float32) + b2_ref[...]


def _mlp(p2, cnts, l1t, b1, l2t, b2):
    return pl.pallas_call(
        _mlp_body,
        out_shape=[
            jax.ShapeDtypeStruct((G, 64), jnp.float32),
            jax.ShapeDtypeStruct((G, 1), jnp.float32),
        ],
    )(p2, cnts, l1t, b1, l2t, b2)


@functools.partial(
    pl.kernel,
    out_type=[
        jax.ShapeDtypeStruct((4 * EP,), jnp.float32),
        jax.ShapeDtypeStruct((NW, 4 * NP), jnp.float32),
    ],
    mesh=_mesh,
    compiler_params=pltpu.CompilerParams(needs_layout_passes=False),
    scratch_types=[
        pltpu.VMEM((4 * NP,), jnp.float32),
        pltpu.VMEM((4 * NP,), jnp.float32),
        pltpu.VMEM((C1,), jnp.int32),
        pltpu.VMEM((C1,), jnp.int32),
        pltpu.VMEM((2, C1), jnp.float32),
        pltpu.VMEM((4, C1), jnp.float32),
    ],
)
def _sc_pass1(dst_ref, src_ref, es_ref, sflat_ref, we_ref, outd_ref,
              stab, denom, dstb, srcb, esb, wbuf):
    c = lax.axis_index("c")
    s = lax.axis_index("s")
    wid = s * 2 + c
    pltpu.sync_copy(sflat_ref, stab)

    def zbody(i, carry):
        denom[pl.ds(i * 16, 16)] = jnp.zeros((16,), jnp.float32)
        return carry
    lax.fori_loop(0, 4 * NP // 16, zbody, 0)

    def chunk(i, carry):
        base = wid * EPW + i * C1
        pltpu.sync_copy(dst_ref.at[pl.ds(base, C1)], dstb)
        pltpu.sync_copy(src_ref.at[pl.ds(base, C1)], srcb)
        pltpu.sync_copy(es_ref.at[pl.ds(base, C1)], esb.at[0])
        pltpu.sync_copy(es_ref.at[pl.ds(EP + base, C1)], esb.at[1])

        def grp(g, carry2):
            dv = dstb[pl.ds(g * 16, 16)]
            sv = srcb[pl.ds(g * 16, 16)]
            for h in range(4):
                if h < 2:
                    al = plsc.load_gather(stab, [dv + h * NP])
                else:
                    al = (plsc.load_gather(stab, [sv + h * NP])
                          + esb[h - 2, pl.ds(g * 16, 16)])
                al = jnp.where(al > 0, al, al * 0.2)
                w = jnp.exp(al)
                plsc.addupdate_scatter(denom, [dv + h * NP], w)
                wbuf[h, pl.ds(g * 16, 16)] = w
            return carry2
        lax.fori_loop(0, C1 // 16, grp, 0)
        for h in range(4):
            pltpu.sync_copy(wbuf.at[h], we_ref.at[pl.ds(h * EP + base, C1)])
        return carry
    lax.fori_loop(0, EPW // C1, chunk, 0)
    pltpu.sync_copy(denom, outd_ref.at[wid])


@functools.partial(
    pl.kernel,
    out_type=jax.ShapeDtypeStruct((EP // C2, 4, C2), jnp.float32),
    mesh=_mesh,
    compiler_params=pltpu.CompilerParams(needs_layout_passes=False),
    scratch_types=[
        pltpu.VMEM((4 * NP,), jnp.float32),
        pltpu.VMEM((C1,), jnp.int32),
        pltpu.VMEM((4, C1), jnp.float32),
        pltpu.VMEM((C1 // C2, 4, C2), jnp.float32),
    ],
)
def _sc_norm(dst_ref, we_ref, inv_ref, al_ref, inv_v, dstb, wb, ab):
    c = lax.axis_index("c")
    s = lax.axis_index("s")
    wid = s * 2 + c
    pltpu.sync_copy(inv_ref, inv_v)

    def chunk(i, carry):
        base = wid * EPW + i * C1
        pltpu.sync_copy(dst_ref.at[pl.ds(base, C1)], dstb)
        for h in range(4):
            pltpu.sync_copy(we_ref.at[pl.ds(h * EP + base, C1)], wb.at[h])

        def grp(g, carry2):
            dv = dstb[pl.ds(g * 16, 16)]
            k = g // 2
            off = (g % 2) * 16
            for h in range(4):
                iv = plsc.load_gather(inv_v, [dv + h * NP])
                ab[k, h, pl.ds(off, 16)] = wb[h, pl.ds(g * 16, 16)] * iv
            return carry2
        lax.fori_loop(0, C1 // 16, grp, 0)
        g0 = base // C2
        for k in range(C1 // C2):
            pltpu.sync_copy(ab.at[k], al_ref.at[g0 + k])
        return carry
    lax.fori_loop(0, EPW // C1, chunk, 0)


@functools.partial(
    pl.kernel,
    out_type=jax.ShapeDtypeStruct((2 * NP, D), jnp.float32),
    mesh=_mesh,
    compiler_params=pltpu.CompilerParams(needs_layout_passes=False),
    scratch_types=[
        pltpu.VMEM((3, 2, C2), jnp.int32),
        pltpu.VMEM((3, 4, C2), jnp.float32),
        pltpu.VMEM((2, C2, HP), jnp.float32),
        pltpu.VMEM((C2, D), jnp.float32),
        pltpu.VMEM((4, 8), jnp.int32),
        pltpu.VMEM_SHARED((NP, D), jnp.float32),
        pltpu.SemaphoreType.DMA((3,)),
        pltpu.SemaphoreType.DMA((2,)),
    ],
)
def _sc_pass2(didx_ref, al_ref, xw_ref, out_ref,
              db, ab, rowbuf, msgbuf, cidx, agg, sem_ia, sem_g):
    c = lax.axis_index("c")
    s = lax.axis_index("s")
    wid = s * 2 + c
    g0 = wid * NCH

    def zrow(r, carry):
        for v in range(8):
            msgbuf[r, pl.ds(v * 16, 16)] = jnp.zeros((16,), jnp.float32)
        return carry
    lax.fori_loop(0, C2, zrow, 0)

    def zagg(k, carry):
        pltpu.sync_copy(msgbuf, agg.at[pl.ds(s * 640 + k * C2, C2)])
        return carry
    lax.fori_loop(0, NP // 16 // C2, zagg, 0)
    plsc.subcore_barrier()

    for sl in range(2):
        pltpu.async_copy(didx_ref.at[g0 + sl], db.at[sl], sem_ia.at[sl])
        pltpu.async_copy(al_ref.at[g0 + sl], ab.at[sl], sem_ia.at[sl])
    pltpu.make_async_copy(didx_ref.at[g0], db.at[0], sem_ia.at[0]).wait()
    pltpu.make_async_copy(al_ref.at[g0], ab.at[0], sem_ia.at[0]).wait()
    pltpu.async_copy(xw_ref.at[db.at[0, 1]], rowbuf.at[0], sem_g.at[0])

    def chunk(i, carry):
        p = lax.rem(i, 2)
        q = lax.rem(i + 1, 2)
        ip = lax.rem(i, 3)
        i1 = lax.rem(i + 1, 3)
        i2 = lax.rem(i + 2, 3)
        nxt2 = jnp.minimum(i + 2, NCH - 1)
        pltpu.async_copy(didx_ref.at[g0 + nxt2], db.at[i2], sem_ia.at[i2])
        pltpu.async_copy(al_ref.at[g0 + nxt2], ab.at[i2], sem_ia.at[i2])
        pltpu.make_async_copy(xw_ref.at[pl.ds(0, C2)], rowbuf.at[0],
                              sem_g.at[p]).wait()

        gi = lax.iota(jnp.int32, 16)
        dmy = jnp.full((16,), NP - 8, jnp.int32)
        plsc.store_scatter(cidx, [gi // 8, gi % 8], dmy)
        plsc.store_scatter(cidx, [(gi + 16) // 8, (gi + 16) % 8], dmy)

        def grp(g, base):
            dv = db[ip, 0, pl.ds(g * 16, 16)]
            pidx = jnp.maximum(g * 16 + gi - 1, 0)
            prev = plsc.load_gather(
                db, [jnp.full((16,), ip, jnp.int32),
                     jnp.zeros((16,), jnp.int32), pidx])
            flags = ((dv != prev) | ((g * 16 + gi) == 0)).astype(jnp.int32)
            cum = plsc.cumsum(flags)
            row = base + cum - 1
            plsc.store_scatter(cidx, [row // 8, row % 8], dv)
            avs = [ab[ip, h, pl.ds(g * 16, 16)] for h in range(4)]
            for j in range(16):
                a0 = avs[0][j]
                a1 = avs[1][j]
                a2 = avs[2][j]
                a3 = avs[3][j]
                rs = row[j]
                fs = flags[j]
                r = g * 16 + j
                for v in range(8):
                    m = (rowbuf[p, r, pl.ds(v * 16, 16)] * a0
                         + rowbuf[p, r, pl.ds(PH + v * 16, 16)] * a1
                         + rowbuf[p, r, pl.ds(2 * PH + v * 16, 16)] * a2
                         + rowbuf[p, r, pl.ds(3 * PH + v * 16, 16)] * a3)
                    mp = msgbuf[rs, pl.ds(v * 16, 16)]
                    msgbuf[rs, pl.ds(v * 16, 16)] = jnp.where(fs == 1, m, m + mp)
            return base + cum[15]
        nrows = lax.fori_loop(0, C2 // 16, grp, 0)

        def scat(k, carry2):
            pltpu.sync_copy(msgbuf.at[pl.ds(k * 8, 8)],
                            agg.at[cidx.at[k]], add=True)
            return carry2
        lax.fori_loop(0, (nrows + 7) // 8, scat, 0)

        nxt1 = jnp.minimum(i + 1, NCH - 1)
        pltpu.make_async_copy(didx_ref.at[g0], db.at[0], sem_ia.at[i1]).wait()
        pltpu.make_async_copy(al_ref.at[g0], ab.at[0], sem_ia.at[i1]).wait()
        pltpu.async_copy(xw_ref.at[db.at[i1, 1]], rowbuf.at[q], sem_g.at[q])
        return carry
    lax.fori_loop(0, NCH, chunk, 0)
    pltpu.make_async_copy(didx_ref.at[g0], db.at[0],
                          sem_ia.at[(NCH + 1) % 3]).wait()
    pltpu.make_async_copy(al_ref.at[g0], ab.at[0],
                          sem_ia.at[(NCH + 1) % 3]).wait()
    pltpu.make_async_copy(xw_ref.at[pl.ds(0, C2)], rowbuf.at[0],
                          sem_g.at[NCH % 2]).wait()
    plsc.subcore_barrier()
    pltpu.sync_copy(agg.at[pl.ds(s * 640, 640)],
                    out_ref.at[pl.ds(c * NP + s * 640, 640)])


@functools.partial(
    pl.kernel,
    out_type=[
        jax.ShapeDtypeStruct((2 * PR, D), jnp.float32),
        jax.ShapeDtypeStruct((NW, 512), jnp.float32),
    ],
    mesh=_mesh,
    compiler_params=pltpu.CompilerParams(needs_layout_passes=False),
    scratch_types=[
        pltpu.VMEM((64, D), jnp.float32),
        pltpu.VMEM((64,), jnp.int32),
        pltpu.VMEM((512,), jnp.float32),
        pltpu.VMEM((24, D), jnp.float32),
        pltpu.VMEM_SHARED((PR, D), jnp.float32),
    ],
)
def _sc_pool(h_ref, b_ref, outp_ref, outc_ref, hbuf, bbuf, cnt, zbuf, pooled):
    c = lax.axis_index("c")
    s = lax.axis_index("s")
    wid = s * 2 + c

    def zc(i, carry):
        cnt[pl.ds(i * 16, 16)] = jnp.zeros((16,), jnp.float32)
        return carry
    lax.fori_loop(0, 32, zc, 0)

    def zrow(r, carry):
        for v in range(8):
            zbuf[r, pl.ds(v * 16, 16)] = jnp.zeros((16,), jnp.float32)
        return carry
    lax.fori_loop(0, 24, zrow, 0)
    pltpu.sync_copy(zbuf, pooled.at[pl.ds(s * 24, 24)])
    plsc.subcore_barrier()

    def chunk(i, carry):
        base = wid * (NP // NW) + i * 64
        pltpu.sync_copy(h_ref.at[pl.ds(base, 64)], hbuf)
        pltpu.sync_copy(b_ref.at[pl.ds(base, 64)], bbuf)
        pltpu.sync_copy(hbuf, pooled.at[bbuf], add=True)

        def grp(g, carry2):
            bv = bbuf[pl.ds(g * 16, 16)]
            plsc.addupdate_scatter(cnt, [bv], jnp.ones((16,), jnp.float32))
            return carry2
        lax.fori_loop(0, 4, grp, 0)
        return carry
    lax.fori_loop(0, NP // NW // 64, chunk, 0)
    plsc.subcore_barrier()
    pltpu.sync_copy(pooled.at[pl.ds(s * 24, 24)],
                    outp_ref.at[pl.ds(c * PR + s * 24, 24)])
    pltpu.sync_copy(cnt, outc_ref.at[wid])


def kernel(x, edge_index, edge_attr, batch, emb, W, W_edge, a, ln_g, ln_b,
           lin1_W, lin1_b, lin2_W, lin2_b):
    n = x.shape[0]
    L = W.shape[0]

    loops = jnp.arange(n, dtype=jnp.int32)
    src = jnp.concatenate([edge_index[0].astype(jnp.int32), loops])
    dst = jnp.concatenate([edge_index[1].astype(jnp.int32), loops])
    pad_e = jnp.full((EP - (E0 + n),), NP - 1, jnp.int32)
    src = jnp.concatenate([src, pad_e])
    dst = jnp.concatenate([dst, pad_e])
    order = jnp.argsort(dst)
    dst = dst[order]
    src = src[order]
    didx = jnp.stack([dst.reshape(EP // C2, C2), src.reshape(EP // C2, C2)],
                     axis=1)

    Wr = W.reshape(L, 2, 2 * PH, D)
    half = jnp.array([0, 1, 0, 1])
    C = jnp.einsum('lhjd,lhj->ldh', Wr[:, half], a)
    C = jnp.concatenate([C, jnp.zeros_like(C)], axis=-1)
    CT = C.transpose(0, 2, 1)
    Wer = W_edge.reshape(L, 2, 2 * PH, D)
    Ce = jnp.einsum('lhjd,lhj->ldh', Wer[:, half[2:]], a[:, 2:])
    Ce_all = Ce.transpose(1, 0, 2).reshape(D, 2 * L)

    es_all = _edge_matmul(edge_attr, Ce_all)
    esT = jnp.zeros((2 * L, EP), jnp.float32).at[:, :E0].set(es_all.T)
    esT = esT[:, order]

    h = jnp.zeros((NP, D), jnp.float32).at[:n].set(emb[x])
    wt_all = W.transpose(0, 2, 1)

    xW, St = _node_matmul(h, wt_all[0], CT[0])
    for l in range(L):
        s_flat = St[:4].reshape(4 * NP)
        es_l = lax.slice(esT, (2 * l, 0), (2 * l + 2, EP)).reshape(2 * EP)
        wE, denoms = _sc_pass1(dst, src, es_l, s_flat)
        inv = 1.0 / (jnp.sum(denoms, axis=0) + 1e-16)
        alphaE = _sc_norm(dst, wE, inv)
        agg2 = _sc_pass2(didx, alphaE, xW)
        agg2 = agg2.reshape(2, NP, D)
        g = ln_g[l].reshape(1, D)
        b = ln_b[l].reshape(1, D)
        if l + 1 < L:
            h, xW, St = _ln_mm(agg2, h, g, b, wt_all[l + 1], CT[l + 1])
        else:
            h = _ln_only(agg2, h, g, b)

    batch_pad = jnp.full((NP,), G, jnp.int32).at[:n].set(batch.astype(jnp.int32))
    p2, cnts = _sc_pool(h, batch_pad)
    hh, out = _mlp(p2, cnts, lin1_W.T, lin1_b.reshape(1, 64),
                   lin2_W.T, lin2_b.reshape(1, 1))
    return out, hh


# speedup vs baseline: 1.0597x; 1.0597x over previous
import functools

import jax
import jax.numpy as jnp
from jax import lax
from jax.experimental import pallas as pl
from jax.experimental.pallas import tpu as pltpu
from jax.experimental.pallas import tpu_sc as plsc

N = 10000
NP = 10240
E0 = 160000
EP = 172032
NW = 32
EPW = EP // NW
C1 = 256
C2 = 32
NCH = EPW // C2
HP = 512
D = 128
HEADS = 4
PH = 128
G = 256
PR = 384

_mesh = plsc.VectorSubcoreMesh(core_axis_name="c", subcore_axis_name="s",
                               num_cores=2, num_subcores=16)


def _node_mm_body(h_ref, wt_ref, ct_ref, xw_ref, st_ref):
    h = h_ref[...]
    xw_ref[...] = jnp.dot(h, wt_ref[...], preferred_element_type=jnp.float32)
    st_ref[...] = lax.dot_general(ct_ref[...], h, (((1,), (1,)), ((), ())),
                                  preferred_element_type=jnp.float32)


def _node_matmul(h, wt, ct, blk=1024):
    return pl.pallas_call(
        _node_mm_body,
        grid=(NP // blk,),
        in_specs=[
            pl.BlockSpec((blk, D), lambda i: (i, 0)),
            pl.BlockSpec((D, HP), lambda i: (0, 0)),
            pl.BlockSpec((8, D), lambda i: (0, 0)),
        ],
        out_specs=[
            pl.BlockSpec((blk, HP), lambda i: (i, 0)),
            pl.BlockSpec((8, blk), lambda i: (0, i)),
        ],
        out_shape=[
            jax.ShapeDtypeStruct((NP, HP), jnp.float32),
            jax.ShapeDtypeStruct((8, NP), jnp.float32),
        ],
    )(h, wt, ct)


def _edge_mm_body(ea_ref, ce_ref, out_ref):
    out_ref[...] = jnp.dot(ea_ref[...], ce_ref[...],
                           preferred_element_type=jnp.float32)


def _edge_matmul(ea, ce, blk=8000):
    e = ea.shape[0]
    return pl.pallas_call(
        _edge_mm_body,
        grid=(e // blk,),
        in_specs=[
            pl.BlockSpec((blk, D), lambda i: (i, 0)),
            pl.BlockSpec((D, 8), lambda i: (0, 0)),
        ],
        out_specs=pl.BlockSpec((blk, 8), lambda i: (i, 0)),
        out_shape=jax.ShapeDtypeStruct((e, 8), jnp.float32),
    )(ea, ce)


def _ln_mm_body(agg_ref, h_ref, g_ref, b_ref, wt_ref, ct_ref,
                hn_ref, xw_ref, st_ref):
    agg = agg_ref[0] + agg_ref[1]
    mu = jnp.mean(agg, axis=-1, keepdims=True)
    var = jnp.mean((agg - mu) ** 2, axis=-1, keepdims=True)
    z = (agg - mu) * lax.rsqrt(var + 1e-5) * g_ref[...] + b_ref[...]
    z = jnp.where(z > 0, z, jnp.exp(z) - 1.0)
    hn = z + h_ref[...]
    hn_ref[...] = hn
    xw_ref[...] = jnp.dot(hn, wt_ref[...], preferred_element_type=jnp.float32)
    st_ref[...] = lax.dot_general(ct_ref[...], hn, (((1,), (1,)), ((), ())),
                                  preferred_element_type=jnp.float32)


def _ln_mm(agg2, h, g, b, wt, ct, blk=1024):
    return pl.pallas_call(
        _ln_mm_body,
        grid=(NP // blk,),
        in_specs=[
            pl.BlockSpec((2, blk, D), lambda i: (0, i, 0)),
            pl.BlockSpec((blk, D), lambda i: (i, 0)),
            pl.BlockSpec((1, D), lambda i: (0, 0)),
            pl.BlockSpec((1, D), lambda i: (0, 0)),
            pl.BlockSpec((D, HP), lambda i: (0, 0)),
            pl.BlockSpec((8, D), lambda i: (0, 0)),
        ],
        out_specs=[
            pl.BlockSpec((blk, D), lambda i: (i, 0)),
            pl.BlockSpec((blk, HP), lambda i: (i, 0)),
            pl.BlockSpec((8, blk), lambda i: (0, i)),
        ],
        out_shape=[
            jax.ShapeDtypeStruct((NP, D), jnp.float32),
            jax.ShapeDtypeStruct((NP, HP), jnp.float32),
            jax.ShapeDtypeStruct((8, NP), jnp.float32),
        ],
    )(agg2, h, g, b, wt, ct)


def _ln_body(agg_ref, h_ref, g_ref, b_ref, hn_ref):
    agg = agg_ref[0] + agg_ref[1]
    mu = jnp.mean(agg, axis=-1, keepdims=True)
    var = jnp.mean((agg - mu) ** 2, axis=-1, keepdims=True)
    z = (agg - mu) * lax.rsqrt(var + 1e-5) * g_ref[...] + b_ref[...]
    z = jnp.where(z > 0, z, jnp.exp(z) - 1.0)
    hn_ref[...] = z + h_ref[...]


def _ln_only(agg2, h, g, b, blk=1024):
    return pl.pallas_call(
        _ln_body,
        grid=(NP // blk,),
        in_specs=[
            pl.BlockSpec((2, blk, D), lambda i: (0, i, 0)),
            pl.BlockSpec((blk, D), lambda i: (i, 0)),
            pl.BlockSpec((1, D), lambda i: (0, 0)),
            pl.BlockSpec((1, D), lambda i: (0, 0)),
        ],
        out_specs=pl.BlockSpec((blk, D), lambda i: (i, 0)),
        out_shape=jax.ShapeDtypeStruct((NP, D), jnp.float32),
    )(agg2, h, g, b)


def _mlp_body(p_ref, c_ref, l1_ref, b1_ref, l2_ref, b2_ref, hh_ref, out_ref):
    p = p_ref[0:PR, :] + p_ref[PR:2 * PR, :]
    po = p[0:G, :]
    cnt = jnp.sum(c_ref[...], axis=0)[0:G]
    pooled = po / jnp.maximum(cnt, 1.0)[:, None]
    hh = jnp.dot(pooled, l1_ref[...], preferred_element_type=jnp.float32) + b1_ref[...]
    hh_ref[...] = hh
    out_ref[...] = jnp.dot(hh, l2_ref[...], preferred_element_type=jnp.float32) + b2_ref[...]


def _mlp(p2, cnts, l1t, b1, l2t, b2):
    return pl.pallas_call(
        _mlp_body,
        out_shape=[
            jax.ShapeDtypeStruct((G, 64), jnp.float32),
            jax.ShapeDtypeStruct((G, 1), jnp.float32),
        ],
    )(p2, cnts, l1t, b1, l2t, b2)


@functools.partial(
    pl.kernel,
    out_type=[
        jax.ShapeDtypeStruct((4 * EP,), jnp.float32),
        jax.ShapeDtypeStruct((NW, 4 * NP), jnp.float32),
    ],
    mesh=_mesh,
    compiler_params=pltpu.CompilerParams(needs_layout_passes=False),
    scratch_types=[
        pltpu.VMEM((4 * NP,), jnp.float32),
        pltpu.VMEM((4 * NP,), jnp.float32),
        pltpu.VMEM((C1,), jnp.int32),
        pltpu.VMEM((C1,), jnp.int32),
        pltpu.VMEM((2, C1), jnp.float32),
        pltpu.VMEM((4, C1), jnp.float32),
    ],
)
def _sc_pass1(dst_ref, src_ref, es_ref, sflat_ref, we_ref, outd_ref,
              stab, denom, dstb, srcb, esb, wbuf):
    c = lax.axis_index("c")
    s = lax.axis_index("s")
    wid = s * 2 + c
    pltpu.sync_copy(sflat_ref, stab)

    def zbody(i, carry):
        denom[pl.ds(i * 16, 16)] = jnp.zeros((16,), jnp.float32)
        return carry
    lax.fori_loop(0, 4 * NP // 16, zbody, 0)

    def chunk(i, carry):
        base = wid * EPW + i * C1
        pltpu.sync_copy(dst_ref.at[pl.ds(base, C1)], dstb)
        pltpu.sync_copy(src_ref.at[pl.ds(base, C1)], srcb)
        pltpu.sync_copy(es_ref.at[pl.ds(base, C1)], esb.at[0])
        pltpu.sync_copy(es_ref.at[pl.ds(EP + base, C1)], esb.at[1])

        def grp(g, carry2):
            dv = dstb[pl.ds(g * 16, 16)]
            sv = srcb[pl.ds(g * 16, 16)]
            for h in range(4):
                if h < 2:
                    al = plsc.load_gather(stab, [dv + h * NP])
                else:
                    al = (plsc.load_gather(stab, [sv + h * NP])
                          + esb[h - 2, pl.ds(g * 16, 16)])
                al = jnp.where(al > 0, al, al * 0.2)
                w = jnp.exp(al)
                plsc.addupdate_scatter(denom, [dv + h * NP], w)
                wbuf[h, pl.ds(g * 16, 16)] = w
            return carry2
        lax.fori_loop(0, C1 // 16, grp, 0)
        for h in range(4):
            pltpu.sync_copy(wbuf.at[h], we_ref.at[pl.ds(h * EP + base, C1)])
        return carry
    lax.fori_loop(0, EPW // C1, chunk, 0)
    pltpu.sync_copy(denom, outd_ref.at[wid])


@functools.partial(
    pl.kernel,
    out_type=jax.ShapeDtypeStruct((EP // C2, 4, C2), jnp.float32),
    mesh=_mesh,
    compiler_params=pltpu.CompilerParams(needs_layout_passes=False),
    scratch_types=[
        pltpu.VMEM((4 * NP,), jnp.float32),
        pltpu.VMEM((C1,), jnp.int32),
        pltpu.VMEM((4, C1), jnp.float32),
        pltpu.VMEM((C1 // C2, 4, C2), jnp.float32),
    ],
)
def _sc_norm(dst_ref, we_ref, inv_ref, al_ref, inv_v, dstb, wb, ab):
    c = lax.axis_index("c")
    s = lax.axis_index("s")
    wid = s * 2 + c
    pltpu.sync_copy(inv_ref, inv_v)

    def chunk(i, carry):
        base = wid * EPW + i * C1
        pltpu.sync_copy(dst_ref.at[pl.ds(base, C1)], dstb)
        for h in range(4):
            pltpu.sync_copy(we_ref.at[pl.ds(h * EP + base, C1)], wb.at[h])

        def grp(g, carry2):
            dv = dstb[pl.ds(g * 16, 16)]
            k = g // 2
            off = (g % 2) * 16
            for h in range(4):
                iv = plsc.load_gather(inv_v, [dv + h * NP])
                ab[k, h, pl.ds(off, 16)] = wb[h, pl.ds(g * 16, 16)] * iv
            return carry2
        lax.fori_loop(0, C1 // 16, grp, 0)
        g0 = base // C2
        for k in range(C1 // C2):
            pltpu.sync_copy(ab.at[k], al_ref.at[g0 + k])
        return carry
    lax.fori_loop(0, EPW // C1, chunk, 0)


@functools.partial(
    pl.kernel,
    out_type=jax.ShapeDtypeStruct((2 * NP, D), jnp.float32),
    mesh=_mesh,
    compiler_params=pltpu.CompilerParams(needs_layout_passes=False),
    scratch_types=[
        pltpu.VMEM((3, 2, C2), jnp.int32),
        pltpu.VMEM((3, 4, C2), jnp.float32),
        pltpu.VMEM((2, C2, HP), jnp.float32),
        pltpu.VMEM((C2, D), jnp.float32),
        pltpu.VMEM((4, 8), jnp.int32),
        pltpu.VMEM_SHARED((NP, D), jnp.float32),
        pltpu.SemaphoreType.DMA((3,)),
        pltpu.SemaphoreType.DMA((2,)),
    ],
)
def _sc_pass2(didx_ref, al_ref, xw_ref, out_ref,
              db, ab, rowbuf, msgbuf, cidx, agg, sem_ia, sem_g):
    c = lax.axis_index("c")
    s = lax.axis_index("s")
    wid = s * 2 + c
    g0 = wid * NCH

    def zrow(r, carry):
        for v in range(8):
            msgbuf[r, pl.ds(v * 16, 16)] = jnp.zeros((16,), jnp.float32)
        return carry
    lax.fori_loop(0, C2, zrow, 0)

    def zagg(k, carry):
        pltpu.sync_copy(msgbuf, agg.at[pl.ds(s * 640 + k * C2, C2)])
        return carry
    lax.fori_loop(0, NP // 16 // C2, zagg, 0)
    plsc.subcore_barrier()

    for sl in range(2):
        pltpu.async_copy(didx_ref.at[g0 + sl], db.at[sl], sem_ia.at[sl])
        pltpu.async_copy(al_ref.at[g0 + sl], ab.at[sl], sem_ia.at[sl])
    pltpu.make_async_copy(didx_ref.at[g0], db.at[0], sem_ia.at[0]).wait()
    pltpu.make_async_copy(al_ref.at[g0], ab.at[0], sem_ia.at[0]).wait()
    pltpu.async_copy(xw_ref.at[db.at[0, 1]], rowbuf.at[0], sem_g.at[0])

    def chunk(i, carry):
        p = lax.rem(i, 2)
        q = lax.rem(i + 1, 2)
        ip = lax.rem(i, 3)
        i1 = lax.rem(i + 1, 3)
        i2 = lax.rem(i + 2, 3)
        nxt2 = jnp.minimum(i + 2, NCH - 1)
        pltpu.async_copy(didx_ref.at[g0 + nxt2], db.at[i2], sem_ia.at[i2])
        pltpu.async_copy(al_ref.at[g0 + nxt2], ab.at[i2], sem_ia.at[i2])
        pltpu.make_async_copy(xw_ref.at[pl.ds(0, C2)], rowbuf.at[0],
                              sem_g.at[p]).wait()

        gi = lax.iota(jnp.int32, 16)
        dmy = jnp.full((16,), NP - 8, jnp.int32)
        plsc.store_scatter(cidx, [gi // 8, gi % 8], dmy)
        plsc.store_scatter(cidx, [(gi + 16) // 8, (gi + 16) % 8], dmy)

        def grp(g, carry2):
            base, acc = carry2
            dv = db[ip, 0, pl.ds(g * 16, 16)]
            pidx = jnp.maximum(g * 16 + gi - 1, 0)
            prev = plsc.load_gather(
                db, [jnp.full((16,), ip, jnp.int32),
                     jnp.zeros((16,), jnp.int32), pidx])
            flags = ((dv != prev) | ((g * 16 + gi) == 0)).astype(jnp.int32)
            cum = plsc.cumsum(flags)
            row = base + cum - 1
            plsc.store_scatter(cidx, [row // 8, row % 8], dv)
            avs = [ab[ip, h, pl.ds(g * 16, 16)] for h in range(4)]
            for j in range(16):
                a0 = avs[0][j]
                a1 = avs[1][j]
                a2 = avs[2][j]
                a3 = avs[3][j]
                rs = row[j]
                fs = flags[j]
                r = g * 16 + j
                nacc = []
                for v in range(8):
                    m = (rowbuf[p, r, pl.ds(v * 16, 16)] * a0
                         + rowbuf[p, r, pl.ds(PH + v * 16, 16)] * a1
                         + rowbuf[p, r, pl.ds(2 * PH + v * 16, 16)] * a2
                         + rowbuf[p, r, pl.ds(3 * PH + v * 16, 16)] * a3)
                    av = jnp.where(fs == 1, m, m + acc[v])
                    msgbuf[rs, pl.ds(v * 16, 16)] = av
                    nacc.append(av)
                acc = tuple(nacc)
            return base + cum[15], acc
        zv = jnp.zeros((16,), jnp.float32)
        nrows, _ = lax.fori_loop(0, C2 // 16, grp,
                                 (0, (zv, zv, zv, zv, zv, zv, zv, zv)))

        def scat(k, carry2):
            pltpu.sync_copy(msgbuf.at[pl.ds(k * 8, 8)],
                            agg.at[cidx.at[k]], add=True)
            return carry2
        lax.fori_loop(0, (nrows + 7) // 8, scat, 0)

        nxt1 = jnp.minimum(i + 1, NCH - 1)
        pltpu.make_async_copy(didx_ref.at[g0], db.at[0], sem_ia.at[i1]).wait()
        pltpu.make_async_copy(al_ref.at[g0], ab.at[0], sem_ia.at[i1]).wait()
        pltpu.async_copy(xw_ref.at[db.at[i1, 1]], rowbuf.at[q], sem_g.at[q])
        return carry
    lax.fori_loop(0, NCH, chunk, 0)
    pltpu.make_async_copy(didx_ref.at[g0], db.at[0],
                          sem_ia.at[(NCH + 1) % 3]).wait()
    pltpu.make_async_copy(al_ref.at[g0], ab.at[0],
                          sem_ia.at[(NCH + 1) % 3]).wait()
    pltpu.make_async_copy(xw_ref.at[pl.ds(0, C2)], rowbuf.at[0],
                          sem_g.at[NCH % 2]).wait()
    plsc.subcore_barrier()
    pltpu.sync_copy(agg.at[pl.ds(s * 640, 640)],
                    out_ref.at[pl.ds(c * NP + s * 640, 640)])


@functools.partial(
    pl.kernel,
    out_type=[
        jax.ShapeDtypeStruct((2 * PR, D), jnp.float32),
        jax.ShapeDtypeStruct((NW, 512), jnp.float32),
    ],
    mesh=_mesh,
    compiler_params=pltpu.CompilerParams(needs_layout_passes=False),
    scratch_types=[
        pltpu.VMEM((64, D), jnp.float32),
        pltpu.VMEM((64,), jnp.int32),
        pltpu.VMEM((512,), jnp.float32),
        pltpu.VMEM((24, D), jnp.float32),
        pltpu.VMEM_SHARED((PR, D), jnp.float32),
    ],
)
def _sc_pool(h_ref, b_ref, outp_ref, outc_ref, hbuf, bbuf, cnt, zbuf, pooled):
    c = lax.axis_index("c")
    s = lax.axis_index("s")
    wid = s * 2 + c

    def zc(i, carry):
        cnt[pl.ds(i * 16, 16)] = jnp.zeros((16,), jnp.float32)
        return carry
    lax.fori_loop(0, 32, zc, 0)

    def zrow(r, carry):
        for v in range(8):
            zbuf[r, pl.ds(v * 16, 16)] = jnp.zeros((16,), jnp.float32)
        return carry
    lax.fori_loop(0, 24, zrow, 0)
    pltpu.sync_copy(zbuf, pooled.at[pl.ds(s * 24, 24)])
    plsc.subcore_barrier()

    def chunk(i, carry):
        base = wid * (NP // NW) + i * 64
        pltpu.sync_copy(h_ref.at[pl.ds(base, 64)], hbuf)
        pltpu.sync_copy(b_ref.at[pl.ds(base, 64)], bbuf)
        pltpu.sync_copy(hbuf, pooled.at[bbuf], add=True)

        def grp(g, carry2):
            bv = bbuf[pl.ds(g * 16, 16)]
            plsc.addupdate_scatter(cnt, [bv], jnp.ones((16,), jnp.float32))
            return carry2
        lax.fori_loop(0, 4, grp, 0)
        return carry
    lax.fori_loop(0, NP // NW // 64, chunk, 0)
    plsc.subcore_barrier()
    pltpu.sync_copy(pooled.at[pl.ds(s * 24, 24)],
                    outp_ref.at[pl.ds(c * PR + s * 24, 24)])
    pltpu.sync_copy(cnt, outc_ref.at[wid])


def kernel(x, edge_index, edge_attr, batch, emb, W, W_edge, a, ln_g, ln_b,
           lin1_W, lin1_b, lin2_W, lin2_b):
    n = x.shape[0]
    L = W.shape[0]

    loops = jnp.arange(n, dtype=jnp.int32)
    src = jnp.concatenate([edge_index[0].astype(jnp.int32), loops])
    dst = jnp.concatenate([edge_index[1].astype(jnp.int32), loops])
    pad_e = jnp.full((EP - (E0 + n),), NP - 1, jnp.int32)
    src = jnp.concatenate([src, pad_e])
    dst = jnp.concatenate([dst, pad_e])
    order = jnp.argsort(dst)
    dst = dst[order]
    src = src[order]
    didx = jnp.stack([dst.reshape(EP // C2, C2), src.reshape(EP // C2, C2)],
                     axis=1)

    Wr = W.reshape(L, 2, 2 * PH, D)
    half = jnp.array([0, 1, 0, 1])
    C = jnp.einsum('lhjd,lhj->ldh', Wr[:, half], a)
    C = jnp.concatenate([C, jnp.zeros_like(C)], axis=-1)
    CT = C.transpose(0, 2, 1)
    Wer = W_edge.reshape(L, 2, 2 * PH, D)
    Ce = jnp.einsum('lhjd,lhj->ldh', Wer[:, half[2:]], a[:, 2:])
    Ce_all = Ce.transpose(1, 0, 2).reshape(D, 2 * L)

    es_all = _edge_matmul(edge_attr, Ce_all)
    esT = jnp.zeros((2 * L, EP), jnp.float32).at[:, :E0].set(es_all.T)
    esT = esT[:, order]

    h = jnp.zeros((NP, D), jnp.float32).at[:n].set(emb[x])
    wt_all = W.transpose(0, 2, 1)

    xW, St = _node_matmul(h, wt_all[0], CT[0])
    for l in range(L):
        s_flat = St[:4].reshape(4 * NP)
        es_l = lax.slice(esT, (2 * l, 0), (2 * l + 2, EP)).reshape(2 * EP)
        wE, denoms = _sc_pass1(dst, src, es_l, s_flat)
        inv = 1.0 / (jnp.sum(denoms, axis=0) + 1e-16)
        alphaE = _sc_norm(dst, wE, inv)
        agg2 = _sc_pass2(didx, alphaE, xW)
        agg2 = agg2.reshape(2, NP, D)
        g = ln_g[l].reshape(1, D)
        b = ln_b[l].reshape(1, D)
        if l + 1 < L:
            h, xW, St = _ln_mm(agg2, h, g, b, wt_all[l + 1], CT[l + 1])
        else:
            h = _ln_only(agg2, h, g, b)

    batch_pad = jnp.full((NP,), G, jnp.int32).at[:n].set(batch.astype(jnp.int32))
    p2, cnts = _sc_pool(h, batch_pad)
    hh, out = _mlp(p2, cnts, lin1_W.T, lin1_b.reshape(1, 64),
                   lin2_W.T, lin2_b.reshape(1, 1))
    return out, hh


# speedup vs baseline: 1.3989x; 1.3201x over previous
import functools

import jax
import jax.numpy as jnp
from jax import lax
from jax.experimental import pallas as pl
from jax.experimental.pallas import tpu as pltpu
from jax.experimental.pallas import tpu_sc as plsc

N = 10000
NP = 10240
E0 = 160000
EP = 172032
NW = 32
EPW = EP // NW
C1 = 256
C2 = 32
NCH = EPW // C2
HP = 512
D = 128
HEADS = 4
PH = 128
G = 256
PR = 384

_mesh = plsc.VectorSubcoreMesh(core_axis_name="c", subcore_axis_name="s",
                               num_cores=2, num_subcores=16)


def _node_mm_body(h_ref, wt_ref, ct_ref, xw_ref, st_ref):
    h = h_ref[...]
    xw_ref[...] = jnp.dot(h, wt_ref[...], preferred_element_type=jnp.float32)
    st_ref[...] = lax.dot_general(ct_ref[...], h, (((1,), (1,)), ((), ())),
                                  preferred_element_type=jnp.float32)


def _node_matmul(h, wt, ct, blk=1024):
    return pl.pallas_call(
        _node_mm_body,
        grid=(NP // blk,),
        in_specs=[
            pl.BlockSpec((blk, D), lambda i: (i, 0)),
            pl.BlockSpec((D, HP), lambda i: (0, 0)),
            pl.BlockSpec((8, D), lambda i: (0, 0)),
        ],
        out_specs=[
            pl.BlockSpec((blk, HP), lambda i: (i, 0)),
            pl.BlockSpec((8, blk), lambda i: (0, i)),
        ],
        out_shape=[
            jax.ShapeDtypeStruct((NP, HP), jnp.float32),
            jax.ShapeDtypeStruct((8, NP), jnp.float32),
        ],
    )(h, wt, ct)


def _edge_mm_body(ea_ref, ce_ref, out_ref):
    out_ref[...] = jnp.dot(ea_ref[...], ce_ref[...],
                           preferred_element_type=jnp.float32)


def _edge_matmul(ea, ce, blk=8000):
    e = ea.shape[0]
    return pl.pallas_call(
        _edge_mm_body,
        grid=(e // blk,),
        in_specs=[
            pl.BlockSpec((blk, D), lambda i: (i, 0)),
            pl.BlockSpec((D, 8), lambda i: (0, 0)),
        ],
        out_specs=pl.BlockSpec((blk, 8), lambda i: (i, 0)),
        out_shape=jax.ShapeDtypeStruct((e, 8), jnp.float32),
    )(ea, ce)


def _ln_mm_body(agg_ref, h_ref, g_ref, b_ref, wt_ref, ct_ref,
                hn_ref, xw_ref, st_ref):
    agg = agg_ref[0] + agg_ref[1]
    mu = jnp.mean(agg, axis=-1, keepdims=True)
    var = jnp.mean((agg - mu) ** 2, axis=-1, keepdims=True)
    z = (agg - mu) * lax.rsqrt(var + 1e-5) * g_ref[...] + b_ref[...]
    z = jnp.where(z > 0, z, jnp.exp(z) - 1.0)
    hn = z + h_ref[...]
    hn_ref[...] = hn
    xw_ref[...] = jnp.dot(hn, wt_ref[...], preferred_element_type=jnp.float32)
    st_ref[...] = lax.dot_general(ct_ref[...], hn, (((1,), (1,)), ((), ())),
                                  preferred_element_type=jnp.float32)


def _ln_mm(agg2, h, g, b, wt, ct, blk=1024):
    return pl.pallas_call(
        _ln_mm_body,
        grid=(NP // blk,),
        in_specs=[
            pl.BlockSpec((2, blk, D), lambda i: (0, i, 0)),
            pl.BlockSpec((blk, D), lambda i: (i, 0)),
            pl.BlockSpec((1, D), lambda i: (0, 0)),
            pl.BlockSpec((1, D), lambda i: (0, 0)),
            pl.BlockSpec((D, HP), lambda i: (0, 0)),
            pl.BlockSpec((8, D), lambda i: (0, 0)),
        ],
        out_specs=[
            pl.BlockSpec((blk, D), lambda i: (i, 0)),
            pl.BlockSpec((blk, HP), lambda i: (i, 0)),
            pl.BlockSpec((8, blk), lambda i: (0, i)),
        ],
        out_shape=[
            jax.ShapeDtypeStruct((NP, D), jnp.float32),
            jax.ShapeDtypeStruct((NP, HP), jnp.float32),
            jax.ShapeDtypeStruct((8, NP), jnp.float32),
        ],
    )(agg2, h, g, b, wt, ct)


def _ln_body(agg_ref, h_ref, g_ref, b_ref, hn_ref):
    agg = agg_ref[0] + agg_ref[1]
    mu = jnp.mean(agg, axis=-1, keepdims=True)
    var = jnp.mean((agg - mu) ** 2, axis=-1, keepdims=True)
    z = (agg - mu) * lax.rsqrt(var + 1e-5) * g_ref[...] + b_ref[...]
    z = jnp.where(z > 0, z, jnp.exp(z) - 1.0)
    hn_ref[...] = z + h_ref[...]


def _ln_only(agg2, h, g, b, blk=1024):
    return pl.pallas_call(
        _ln_body,
        grid=(NP // blk,),
        in_specs=[
            pl.BlockSpec((2, blk, D), lambda i: (0, i, 0)),
            pl.BlockSpec((blk, D), lambda i: (i, 0)),
            pl.BlockSpec((1, D), lambda i: (0, 0)),
            pl.BlockSpec((1, D), lambda i: (0, 0)),
        ],
        out_specs=pl.BlockSpec((blk, D), lambda i: (i, 0)),
        out_shape=jax.ShapeDtypeStruct((NP, D), jnp.float32),
    )(agg2, h, g, b)


def _mlp_body(p_ref, c_ref, l1_ref, b1_ref, l2_ref, b2_ref, hh_ref, out_ref):
    p = p_ref[0:PR, :] + p_ref[PR:2 * PR, :]
    po = p[0:G, :]
    cnt = jnp.sum(c_ref[...], axis=0)[0:G]
    pooled = po / jnp.maximum(cnt, 1.0)[:, None]
    hh = jnp.dot(pooled, l1_ref[...], preferred_element_type=jnp.float32) + b1_ref[...]
    hh_ref[...] = hh
    out_ref[...] = jnp.dot(hh, l2_ref[...], preferred_element_type=jnp.float32) + b2_ref[...]


def _mlp(p2, cnts, l1t, b1, l2t, b2):
    return pl.pallas_call(
        _mlp_body,
        out_shape=[
            jax.ShapeDtypeStruct((G, 64), jnp.float32),
            jax.ShapeDtypeStruct((G, 1), jnp.float32),
        ],
    )(p2, cnts, l1t, b1, l2t, b2)


@functools.partial(
    pl.kernel,
    out_type=[
        jax.ShapeDtypeStruct((4 * EP,), jnp.float32),
        jax.ShapeDtypeStruct((NW, 4 * NP), jnp.float32),
    ],
    mesh=_mesh,
    compiler_params=pltpu.CompilerParams(needs_layout_passes=False),
    scratch_types=[
        pltpu.VMEM((4 * NP,), jnp.float32),
        pltpu.VMEM((4 * NP,), jnp.float32),
        pltpu.VMEM((C1,), jnp.int32),
        pltpu.VMEM((C1,), jnp.int32),
        pltpu.VMEM((2, C1), jnp.float32),
        pltpu.VMEM((4, C1), jnp.float32),
    ],
)
def _sc_pass1(dst_ref, src_ref, es_ref, sflat_ref, we_ref, outd_ref,
              stab, denom, dstb, srcb, esb, wbuf):
    c = lax.axis_index("c")
    s = lax.axis_index("s")
    wid = s * 2 + c
    pltpu.sync_copy(sflat_ref, stab)

    def zbody(i, carry):
        denom[pl.ds(i * 16, 16)] = jnp.zeros((16,), jnp.float32)
        return carry
    lax.fori_loop(0, 4 * NP // 16, zbody, 0)

    def chunk(i, carry):
        base = wid * EPW + i * C1
        pltpu.sync_copy(dst_ref.at[pl.ds(base, C1)], dstb)
        pltpu.sync_copy(src_ref.at[pl.ds(base, C1)], srcb)
        pltpu.sync_copy(es_ref.at[pl.ds(base, C1)], esb.at[0])
        pltpu.sync_copy(es_ref.at[pl.ds(EP + base, C1)], esb.at[1])

        def grp(g, carry2):
            dv = dstb[pl.ds(g * 16, 16)]
            sv = srcb[pl.ds(g * 16, 16)]
            for h in range(4):
                if h < 2:
                    al = plsc.load_gather(stab, [dv + h * NP])
                else:
                    al = (plsc.load_gather(stab, [sv + h * NP])
                          + esb[h - 2, pl.ds(g * 16, 16)])
                al = jnp.where(al > 0, al, al * 0.2)
                w = jnp.exp(al)
                plsc.addupdate_scatter(denom, [dv + h * NP], w)
                wbuf[h, pl.ds(g * 16, 16)] = w
            return carry2
        lax.fori_loop(0, C1 // 16, grp, 0)
        for h in range(4):
            pltpu.sync_copy(wbuf.at[h], we_ref.at[pl.ds(h * EP + base, C1)])
        return carry
    lax.fori_loop(0, EPW // C1, chunk, 0)
    pltpu.sync_copy(denom, outd_ref.at[wid])


@functools.partial(
    pl.kernel,
    out_type=jax.ShapeDtypeStruct((EP // C2, 4, C2), jnp.float32),
    mesh=_mesh,
    compiler_params=pltpu.CompilerParams(needs_layout_passes=False),
    scratch_types=[
        pltpu.VMEM((4 * NP,), jnp.float32),
        pltpu.VMEM((C1,), jnp.int32),
        pltpu.VMEM((4, C1), jnp.float32),
        pltpu.VMEM((C1 // C2, 4, C2), jnp.float32),
    ],
)
def _sc_norm(dst_ref, we_ref, inv_ref, al_ref, inv_v, dstb, wb, ab):
    c = lax.axis_index("c")
    s = lax.axis_index("s")
    wid = s * 2 + c
    pltpu.sync_copy(inv_ref, inv_v)

    def chunk(i, carry):
        base = wid * EPW + i * C1
        pltpu.sync_copy(dst_ref.at[pl.ds(base, C1)], dstb)
        for h in range(4):
            pltpu.sync_copy(we_ref.at[pl.ds(h * EP + base, C1)], wb.at[h])

        def grp(g, carry2):
            dv = dstb[pl.ds(g * 16, 16)]
            k = g // 2
            off = (g % 2) * 16
            for h in range(4):
                iv = plsc.load_gather(inv_v, [dv + h * NP])
                ab[k, h, pl.ds(off, 16)] = wb[h, pl.ds(g * 16, 16)] * iv
            return carry2
        lax.fori_loop(0, C1 // 16, grp, 0)
        g0 = base // C2
        for k in range(C1 // C2):
            pltpu.sync_copy(ab.at[k], al_ref.at[g0 + k])
        return carry
    lax.fori_loop(0, EPW // C1, chunk, 0)


@functools.partial(
    pl.kernel,
    out_type=jax.ShapeDtypeStruct((2 * NP, D), jnp.float32),
    mesh=_mesh,
    compiler_params=pltpu.CompilerParams(needs_layout_passes=False),
    scratch_types=[
        pltpu.VMEM((3, 2, C2), jnp.int32),
        pltpu.VMEM((3, 4, C2), jnp.float32),
        pltpu.VMEM((2, C2, HP), jnp.float32),
        pltpu.VMEM((C2, D), jnp.float32),
        pltpu.VMEM((4, 8), jnp.int32),
        pltpu.VMEM_SHARED((NP, D), jnp.float32),
        pltpu.SemaphoreType.DMA((3,)),
        pltpu.SemaphoreType.DMA((2,)),
    ],
)
def _sc_pass2(didx_ref, al_ref, xw_ref, out_ref,
              db, ab, rowbuf, msgbuf, cidx, agg, sem_ia, sem_g):
    c = lax.axis_index("c")
    s = lax.axis_index("s")
    wid = s * 2 + c
    g0 = wid * NCH

    def zrow(r, carry):
        for v in range(8):
            msgbuf[r, pl.ds(v * 16, 16)] = jnp.zeros((16,), jnp.float32)
        return carry
    lax.fori_loop(0, C2, zrow, 0)

    def zagg(k, carry):
        pltpu.sync_copy(msgbuf, agg.at[pl.ds(s * 640 + k * C2, C2)])
        return carry
    lax.fori_loop(0, NP // 16 // C2, zagg, 0)
    plsc.subcore_barrier()

    for sl in range(2):
        pltpu.async_copy(didx_ref.at[g0 + sl], db.at[sl], sem_ia.at[sl])
        pltpu.async_copy(al_ref.at[g0 + sl], ab.at[sl], sem_ia.at[sl])
    pltpu.make_async_copy(didx_ref.at[g0], db.at[0], sem_ia.at[0]).wait()
    pltpu.make_async_copy(al_ref.at[g0], ab.at[0], sem_ia.at[0]).wait()
    pltpu.async_copy(xw_ref.at[db.at[0, 1]], rowbuf.at[0], sem_g.at[0])

    def chunk(i, carry):
        p = lax.rem(i, 2)
        q = lax.rem(i + 1, 2)
        ip = lax.rem(i, 3)
        i1 = lax.rem(i + 1, 3)
        i2 = lax.rem(i + 2, 3)
        nxt2 = jnp.minimum(i + 2, NCH - 1)
        pltpu.async_copy(didx_ref.at[g0 + nxt2], db.at[i2], sem_ia.at[i2])
        pltpu.async_copy(al_ref.at[g0 + nxt2], ab.at[i2], sem_ia.at[i2])
        pltpu.make_async_copy(xw_ref.at[pl.ds(0, C2)], rowbuf.at[0],
                              sem_g.at[p]).wait()

        def grp(g, carry2):
            avs = [ab[ip, h, pl.ds(g * 16, 16)] for h in range(4)]
            for j in range(16):
                a0 = avs[0][j]
                a1 = avs[1][j]
                a2 = avs[2][j]
                a3 = avs[3][j]
                r = g * 16 + j
                for v in range(8):
                    m = (rowbuf[p, r, pl.ds(v * 16, 16)] * a0
                         + rowbuf[p, r, pl.ds(PH + v * 16, 16)] * a1
                         + rowbuf[p, r, pl.ds(2 * PH + v * 16, 16)] * a2
                         + rowbuf[p, r, pl.ds(3 * PH + v * 16, 16)] * a3)
                    msgbuf[r, pl.ds(v * 16, 16)] = m
            return carry2
        lax.fori_loop(0, C2 // 16, grp, 0)
        pltpu.sync_copy(msgbuf, agg.at[db.at[ip, 0]], add=True)

        nxt1 = jnp.minimum(i + 1, NCH - 1)
        pltpu.make_async_copy(didx_ref.at[g0], db.at[0], sem_ia.at[i1]).wait()
        pltpu.make_async_copy(al_ref.at[g0], ab.at[0], sem_ia.at[i1]).wait()
        pltpu.async_copy(xw_ref.at[db.at[i1, 1]], rowbuf.at[q], sem_g.at[q])
        return carry
    lax.fori_loop(0, NCH, chunk, 0)
    pltpu.make_async_copy(didx_ref.at[g0], db.at[0],
                          sem_ia.at[(NCH + 1) % 3]).wait()
    pltpu.make_async_copy(al_ref.at[g0], ab.at[0],
                          sem_ia.at[(NCH + 1) % 3]).wait()
    pltpu.make_async_copy(xw_ref.at[pl.ds(0, C2)], rowbuf.at[0],
                          sem_g.at[NCH % 2]).wait()
    plsc.subcore_barrier()
    pltpu.sync_copy(agg.at[pl.ds(s * 640, 640)],
                    out_ref.at[pl.ds(c * NP + s * 640, 640)])


@functools.partial(
    pl.kernel,
    out_type=[
        jax.ShapeDtypeStruct((2 * PR, D), jnp.float32),
        jax.ShapeDtypeStruct((NW, 512), jnp.float32),
    ],
    mesh=_mesh,
    compiler_params=pltpu.CompilerParams(needs_layout_passes=False),
    scratch_types=[
        pltpu.VMEM((64, D), jnp.float32),
        pltpu.VMEM((64,), jnp.int32),
        pltpu.VMEM((512,), jnp.float32),
        pltpu.VMEM((24, D), jnp.float32),
        pltpu.VMEM_SHARED((PR, D), jnp.float32),
    ],
)
def _sc_pool(h_ref, b_ref, outp_ref, outc_ref, hbuf, bbuf, cnt, zbuf, pooled):
    c = lax.axis_index("c")
    s = lax.axis_index("s")
    wid = s * 2 + c

    def zc(i, carry):
        cnt[pl.ds(i * 16, 16)] = jnp.zeros((16,), jnp.float32)
        return carry
    lax.fori_loop(0, 32, zc, 0)

    def zrow(r, carry):
        for v in range(8):
            zbuf[r, pl.ds(v * 16, 16)] = jnp.zeros((16,), jnp.float32)
        return carry
    lax.fori_loop(0, 24, zrow, 0)
    pltpu.sync_copy(zbuf, pooled.at[pl.ds(s * 24, 24)])
    plsc.subcore_barrier()

    def chunk(i, carry):
        base = wid * (NP // NW) + i * 64
        pltpu.sync_copy(h_ref.at[pl.ds(base, 64)], hbuf)
        pltpu.sync_copy(b_ref.at[pl.ds(base, 64)], bbuf)
        pltpu.sync_copy(hbuf, pooled.at[bbuf], add=True)

        def grp(g, carry2):
            bv = bbuf[pl.ds(g * 16, 16)]
            plsc.addupdate_scatter(cnt, [bv], jnp.ones((16,), jnp.float32))
            return carry2
        lax.fori_loop(0, 4, grp, 0)
        return carry
    lax.fori_loop(0, NP // NW // 64, chunk, 0)
    plsc.subcore_barrier()
    pltpu.sync_copy(pooled.at[pl.ds(s * 24, 24)],
                    outp_ref.at[pl.ds(c * PR + s * 24, 24)])
    pltpu.sync_copy(cnt, outc_ref.at[wid])


def kernel(x, edge_index, edge_attr, batch, emb, W, W_edge, a, ln_g, ln_b,
           lin1_W, lin1_b, lin2_W, lin2_b):
    n = x.shape[0]
    L = W.shape[0]

    loops = jnp.arange(n, dtype=jnp.int32)
    src = jnp.concatenate([edge_index[0].astype(jnp.int32), loops])
    dst = jnp.concatenate([edge_index[1].astype(jnp.int32), loops])
    pad_e = jnp.full((EP - (E0 + n),), NP - 1, jnp.int32)
    src = jnp.concatenate([src, pad_e])
    dst = jnp.concatenate([dst, pad_e])
    order = jnp.argsort(dst)
    dst = dst[order]
    src = src[order]
    didx = jnp.stack([dst.reshape(EP // C2, C2), src.reshape(EP // C2, C2)],
                     axis=1)

    Wr = W.reshape(L, 2, 2 * PH, D)
    half = jnp.array([0, 1, 0, 1])
    C = jnp.einsum('lhjd,lhj->ldh', Wr[:, half], a)
    C = jnp.concatenate([C, jnp.zeros_like(C)], axis=-1)
    CT = C.transpose(0, 2, 1)
    Wer = W_edge.reshape(L, 2, 2 * PH, D)
    Ce = jnp.einsum('lhjd,lhj->ldh', Wer[:, half[2:]], a[:, 2:])
    Ce_all = Ce.transpose(1, 0, 2).reshape(D, 2 * L)

    es_all = _edge_matmul(edge_attr, Ce_all)
    esT = jnp.zeros((2 * L, EP), jnp.float32).at[:, :E0].set(es_all.T)
    esT = esT[:, order]

    h = jnp.zeros((NP, D), jnp.float32).at[:n].set(emb[x])
    wt_all = W.transpose(0, 2, 1)

    xW, St = _node_matmul(h, wt_all[0], CT[0])
    for l in range(L):
        s_flat = St[:4].reshape(4 * NP)
        es_l = lax.slice(esT, (2 * l, 0), (2 * l + 2, EP)).reshape(2 * EP)
        wE, denoms = _sc_pass1(dst, src, es_l, s_flat)
        inv = 1.0 / (jnp.sum(denoms, axis=0) + 1e-16)
        alphaE = _sc_norm(dst, wE, inv)
        agg2 = _sc_pass2(didx, alphaE, xW)
        agg2 = agg2.reshape(2, NP, D)
        g = ln_g[l].reshape(1, D)
        b = ln_b[l].reshape(1, D)
        if l + 1 < L:
            h, xW, St = _ln_mm(agg2, h, g, b, wt_all[l + 1], CT[l + 1])
        else:
            h = _ln_only(agg2, h, g, b)

    batch_pad = jnp.full((NP,), G, jnp.int32).at[:n].set(batch.astype(jnp.int32))
    p2, cnts = _sc_pool(h, batch_pad)
    hh, out = _mlp(p2, cnts, lin1_W.T, lin1_b.reshape(1, 64),
                   lin2_W.T, lin2_b.reshape(1, 1))
    return out, hh


# speedup vs baseline: 1.5798x; 1.1293x over previous
import functools

import jax
import jax.numpy as jnp
from jax import lax
from jax.experimental import pallas as pl
from jax.experimental.pallas import tpu as pltpu
from jax.experimental.pallas import tpu_sc as plsc

N = 10000
NP = 10240
E0 = 160000
EP = 172032
NW = 32
EPW = EP // NW
C1 = 256
C2 = 32
NCH = EPW // C2
HP = 512
D = 128
HEADS = 4
PH = 128
G = 256
PR = 384

_mesh = plsc.VectorSubcoreMesh(core_axis_name="c", subcore_axis_name="s",
                               num_cores=2, num_subcores=16)


def _node_mm_body(h_ref, wt_ref, ct_ref, xw_ref, st_ref):
    h = h_ref[...]
    xw_ref[...] = jnp.dot(h, wt_ref[...], preferred_element_type=jnp.float32)
    st_ref[...] = lax.dot_general(ct_ref[...], h, (((1,), (1,)), ((), ())),
                                  preferred_element_type=jnp.float32)


def _node_matmul(h, wt, ct, blk=1024):
    return pl.pallas_call(
        _node_mm_body,
        grid=(NP // blk,),
        in_specs=[
            pl.BlockSpec((blk, D), lambda i: (i, 0)),
            pl.BlockSpec((D, HP), lambda i: (0, 0)),
            pl.BlockSpec((8, D), lambda i: (0, 0)),
        ],
        out_specs=[
            pl.BlockSpec((blk, HP), lambda i: (i, 0)),
            pl.BlockSpec((8, blk), lambda i: (0, i)),
        ],
        out_shape=[
            jax.ShapeDtypeStruct((NP, HP), jnp.float32),
            jax.ShapeDtypeStruct((8, NP), jnp.float32),
        ],
    )(h, wt, ct)


def _edge_mm_body(ea_ref, ce_ref, out_ref):
    out_ref[...] = jnp.dot(ea_ref[...], ce_ref[...],
                           preferred_element_type=jnp.float32)


def _edge_matmul(ea, ce, blk=8000):
    e = ea.shape[0]
    return pl.pallas_call(
        _edge_mm_body,
        grid=(e // blk,),
        in_specs=[
            pl.BlockSpec((blk, D), lambda i: (i, 0)),
            pl.BlockSpec((D, 8), lambda i: (0, 0)),
        ],
        out_specs=pl.BlockSpec((blk, 8), lambda i: (i, 0)),
        out_shape=jax.ShapeDtypeStruct((e, 8), jnp.float32),
    )(ea, ce)


def _ln_mm_body(agg_ref, h_ref, g_ref, b_ref, wt_ref, ct_ref,
                hn_ref, xw_ref, st_ref):
    agg = agg_ref[0] + agg_ref[1]
    mu = jnp.mean(agg, axis=-1, keepdims=True)
    var = jnp.mean((agg - mu) ** 2, axis=-1, keepdims=True)
    z = (agg - mu) * lax.rsqrt(var + 1e-5) * g_ref[...] + b_ref[...]
    z = jnp.where(z > 0, z, jnp.exp(z) - 1.0)
    hn = z + h_ref[...]
    hn_ref[...] = hn
    xw_ref[...] = jnp.dot(hn, wt_ref[...], preferred_element_type=jnp.float32)
    st_ref[...] = lax.dot_general(ct_ref[...], hn, (((1,), (1,)), ((), ())),
                                  preferred_element_type=jnp.float32)


def _ln_mm(agg2, h, g, b, wt, ct, blk=1024):
    return pl.pallas_call(
        _ln_mm_body,
        grid=(NP // blk,),
        in_specs=[
            pl.BlockSpec((2, blk, D), lambda i: (0, i, 0)),
            pl.BlockSpec((blk, D), lambda i: (i, 0)),
            pl.BlockSpec((1, D), lambda i: (0, 0)),
            pl.BlockSpec((1, D), lambda i: (0, 0)),
            pl.BlockSpec((D, HP), lambda i: (0, 0)),
            pl.BlockSpec((8, D), lambda i: (0, 0)),
        ],
        out_specs=[
            pl.BlockSpec((blk, D), lambda i: (i, 0)),
            pl.BlockSpec((blk, HP), lambda i: (i, 0)),
            pl.BlockSpec((8, blk), lambda i: (0, i)),
        ],
        out_shape=[
            jax.ShapeDtypeStruct((NP, D), jnp.float32),
            jax.ShapeDtypeStruct((NP, HP), jnp.float32),
            jax.ShapeDtypeStruct((8, NP), jnp.float32),
        ],
    )(agg2, h, g, b, wt, ct)


def _ln_body(agg_ref, h_ref, g_ref, b_ref, hn_ref):
    agg = agg_ref[0] + agg_ref[1]
    mu = jnp.mean(agg, axis=-1, keepdims=True)
    var = jnp.mean((agg - mu) ** 2, axis=-1, keepdims=True)
    z = (agg - mu) * lax.rsqrt(var + 1e-5) * g_ref[...] + b_ref[...]
    z = jnp.where(z > 0, z, jnp.exp(z) - 1.0)
    hn_ref[...] = z + h_ref[...]


def _ln_only(agg2, h, g, b, blk=1024):
    return pl.pallas_call(
        _ln_body,
        grid=(NP // blk,),
        in_specs=[
            pl.BlockSpec((2, blk, D), lambda i: (0, i, 0)),
            pl.BlockSpec((blk, D), lambda i: (i, 0)),
            pl.BlockSpec((1, D), lambda i: (0, 0)),
            pl.BlockSpec((1, D), lambda i: (0, 0)),
        ],
        out_specs=pl.BlockSpec((blk, D), lambda i: (i, 0)),
        out_shape=jax.ShapeDtypeStruct((NP, D), jnp.float32),
    )(agg2, h, g, b)


def _mlp_body(p_ref, c_ref, l1_ref, b1_ref, l2_ref, b2_ref, hh_ref, out_ref):
    p = p_ref[0:PR, :] + p_ref[PR:2 * PR, :]
    po = p[0:G, :]
    cnt = jnp.sum(c_ref[...], axis=0)[0:G]
    pooled = po / jnp.maximum(cnt, 1.0)[:, None]
    hh = jnp.dot(pooled, l1_ref[...], preferred_element_type=jnp.float32) + b1_ref[...]
    hh_ref[...] = hh
    out_ref[...] = jnp.dot(hh, l2_ref[...], preferred_element_type=jnp.float32) + b2_ref[...]


def _mlp(p2, cnts, l1t, b1, l2t, b2):
    return pl.pallas_call(
        _mlp_body,
        out_shape=[
            jax.ShapeDtypeStruct((G, 64), jnp.float32),
            jax.ShapeDtypeStruct((G, 1), jnp.float32),
        ],
    )(p2, cnts, l1t, b1, l2t, b2)


@functools.partial(
    pl.kernel,
    out_type=[
        jax.ShapeDtypeStruct((4 * EP,), jnp.float32),
        jax.ShapeDtypeStruct((NW, 4 * NP), jnp.float32),
    ],
    mesh=_mesh,
    compiler_params=pltpu.CompilerParams(needs_layout_passes=False),
    scratch_types=[
        pltpu.VMEM((4 * NP,), jnp.float32),
        pltpu.VMEM((4 * NP,), jnp.float32),
        pltpu.VMEM((C1,), jnp.int32),
        pltpu.VMEM((C1,), jnp.int32),
        pltpu.VMEM((2, C1), jnp.float32),
        pltpu.VMEM((4, C1), jnp.float32),
    ],
)
def _sc_pass1(dst_ref, src_ref, es_ref, sflat_ref, we_ref, outd_ref,
              stab, denom, dstb, srcb, esb, wbuf):
    c = lax.axis_index("c")
    s = lax.axis_index("s")
    wid = s * 2 + c
    pltpu.sync_copy(sflat_ref, stab)

    def zbody(i, carry):
        denom[pl.ds(i * 16, 16)] = jnp.zeros((16,), jnp.float32)
        return carry
    lax.fori_loop(0, 4 * NP // 16, zbody, 0)

    def chunk(i, carry):
        base = wid * EPW + i * C1
        pltpu.sync_copy(dst_ref.at[pl.ds(base, C1)], dstb)
        pltpu.sync_copy(src_ref.at[pl.ds(base, C1)], srcb)
        pltpu.sync_copy(es_ref.at[pl.ds(base, C1)], esb.at[0])
        pltpu.sync_copy(es_ref.at[pl.ds(EP + base, C1)], esb.at[1])

        def grp(g, carry2):
            dv = dstb[pl.ds(g * 16, 16)]
            sv = srcb[pl.ds(g * 16, 16)]
            for h in range(4):
                if h < 2:
                    al = plsc.load_gather(stab, [dv + h * NP])
                else:
                    al = (plsc.load_gather(stab, [sv + h * NP])
                          + esb[h - 2, pl.ds(g * 16, 16)])
                al = jnp.where(al > 0, al, al * 0.2)
                w = jnp.exp(al)
                plsc.addupdate_scatter(denom, [dv + h * NP], w)
                wbuf[h, pl.ds(g * 16, 16)] = w
            return carry2
        lax.fori_loop(0, C1 // 16, grp, 0)
        for h in range(4):
            pltpu.sync_copy(wbuf.at[h], we_ref.at[pl.ds(h * EP + base, C1)])
        return carry
    lax.fori_loop(0, EPW // C1, chunk, 0)
    pltpu.sync_copy(denom, outd_ref.at[wid])


@functools.partial(
    pl.kernel,
    out_type=jax.ShapeDtypeStruct((EP // C2, 4, C2), jnp.float32),
    mesh=_mesh,
    compiler_params=pltpu.CompilerParams(needs_layout_passes=False),
    scratch_types=[
        pltpu.VMEM((4 * NP,), jnp.float32),
        pltpu.VMEM((C1,), jnp.int32),
        pltpu.VMEM((4, C1), jnp.float32),
        pltpu.VMEM((C1 // C2, 4, C2), jnp.float32),
    ],
)
def _sc_norm(dst_ref, we_ref, inv_ref, al_ref, inv_v, dstb, wb, ab):
    c = lax.axis_index("c")
    s = lax.axis_index("s")
    wid = s * 2 + c
    pltpu.sync_copy(inv_ref, inv_v)

    def chunk(i, carry):
        base = wid * EPW + i * C1
        pltpu.sync_copy(dst_ref.at[pl.ds(base, C1)], dstb)
        for h in range(4):
            pltpu.sync_copy(we_ref.at[pl.ds(h * EP + base, C1)], wb.at[h])

        def grp(g, carry2):
            dv = dstb[pl.ds(g * 16, 16)]
            k = g // 2
            off = (g % 2) * 16
            for h in range(4):
                iv = plsc.load_gather(inv_v, [dv + h * NP])
                ab[k, h, pl.ds(off, 16)] = wb[h, pl.ds(g * 16, 16)] * iv
            return carry2
        lax.fori_loop(0, C1 // 16, grp, 0)
        g0 = base // C2
        for k in range(C1 // C2):
            pltpu.sync_copy(ab.at[k], al_ref.at[g0 + k])
        return carry
    lax.fori_loop(0, EPW // C1, chunk, 0)


@functools.partial(
    pl.kernel,
    out_type=jax.ShapeDtypeStruct((2 * NP, D), jnp.float32),
    mesh=_mesh,
    compiler_params=pltpu.CompilerParams(needs_layout_passes=False),
    scratch_types=[
        pltpu.VMEM((4, 2, C2), jnp.int32),
        pltpu.VMEM((4, 4, C2), jnp.float32),
        pltpu.VMEM((2, C2, HP), jnp.float32),
        pltpu.VMEM((2, C2, D), jnp.float32),
        pltpu.VMEM_SHARED((NP, D), jnp.float32),
        pltpu.SemaphoreType.DMA((4,)),
        pltpu.SemaphoreType.DMA((2,)),
        pltpu.SemaphoreType.DMA((2,)),
    ],
)
def _sc_pass2(didx_ref, al_ref, xw_ref, out_ref,
              db, ab, rowbuf, msgbuf, agg, sem_ia, sem_g, sem_s):
    c = lax.axis_index("c")
    s = lax.axis_index("s")
    wid = s * 2 + c
    g0 = wid * NCH

    def zrow(r, carry):
        for v in range(8):
            msgbuf[0, r, pl.ds(v * 16, 16)] = jnp.zeros((16,), jnp.float32)
        return carry
    lax.fori_loop(0, C2, zrow, 0)

    def zagg(k, carry):
        pltpu.sync_copy(msgbuf.at[0], agg.at[pl.ds(s * 640 + k * C2, C2)])
        return carry
    lax.fori_loop(0, NP // 16 // C2, zagg, 0)
    plsc.subcore_barrier()

    for sl in range(2):
        pltpu.async_copy(didx_ref.at[g0 + sl], db.at[sl], sem_ia.at[sl])
        pltpu.async_copy(al_ref.at[g0 + sl], ab.at[sl], sem_ia.at[sl])
    pltpu.make_async_copy(didx_ref.at[g0], db.at[0], sem_ia.at[0]).wait()
    pltpu.make_async_copy(al_ref.at[g0], ab.at[0], sem_ia.at[0]).wait()
    pltpu.async_copy(xw_ref.at[db.at[0, 1]], rowbuf.at[0], sem_g.at[0])

    def chunk(i, carry):
        p = lax.rem(i, 2)
        q = lax.rem(i + 1, 2)
        ip = lax.rem(i, 4)
        i1 = lax.rem(i + 1, 4)
        i2 = lax.rem(i + 2, 4)
        nxt2 = jnp.minimum(i + 2, NCH - 1)

        @pl.when(i >= 2)
        def _():
            pltpu.make_async_copy(out_ref.at[pl.ds(0, C2)], msgbuf.at[p],
                                  sem_s.at[p]).wait()
        pltpu.async_copy(didx_ref.at[g0 + nxt2], db.at[i2], sem_ia.at[i2])
        pltpu.async_copy(al_ref.at[g0 + nxt2], ab.at[i2], sem_ia.at[i2])
        pltpu.make_async_copy(xw_ref.at[pl.ds(0, C2)], rowbuf.at[0],
                              sem_g.at[p]).wait()

        def grp(g, carry2):
            avs = [ab[ip, h, pl.ds(g * 16, 16)] for h in range(4)]
            for j in range(16):
                a0 = avs[0][j]
                a1 = avs[1][j]
                a2 = avs[2][j]
                a3 = avs[3][j]
                r = g * 16 + j
                for v in range(8):
                    m = (rowbuf[p, r, pl.ds(v * 16, 16)] * a0
                         + rowbuf[p, r, pl.ds(PH + v * 16, 16)] * a1
                         + rowbuf[p, r, pl.ds(2 * PH + v * 16, 16)] * a2
                         + rowbuf[p, r, pl.ds(3 * PH + v * 16, 16)] * a3)
                    msgbuf[p, r, pl.ds(v * 16, 16)] = m
            return carry2
        lax.fori_loop(0, C2 // 16, grp, 0)
        pltpu.async_copy(msgbuf.at[p], agg.at[db.at[ip, 0]], sem_s.at[p],
                         add=True)

        nxt1 = jnp.minimum(i + 1, NCH - 1)
        pltpu.make_async_copy(didx_ref.at[g0], db.at[0], sem_ia.at[i1]).wait()
        pltpu.make_async_copy(al_ref.at[g0], ab.at[0], sem_ia.at[i1]).wait()
        pltpu.async_copy(xw_ref.at[db.at[i1, 1]], rowbuf.at[q], sem_g.at[q])
        return carry
    lax.fori_loop(0, NCH, chunk, 0)
    pltpu.make_async_copy(didx_ref.at[g0], db.at[0],
                          sem_ia.at[(NCH + 1) % 4]).wait()
    pltpu.make_async_copy(al_ref.at[g0], ab.at[0],
                          sem_ia.at[(NCH + 1) % 4]).wait()
    pltpu.make_async_copy(out_ref.at[pl.ds(0, C2)], msgbuf.at[0],
                          sem_s.at[0]).wait()
    pltpu.make_async_copy(out_ref.at[pl.ds(0, C2)], msgbuf.at[1],
                          sem_s.at[1]).wait()
    pltpu.make_async_copy(xw_ref.at[pl.ds(0, C2)], rowbuf.at[0],
                          sem_g.at[NCH % 2]).wait()
    plsc.subcore_barrier()
    pltpu.sync_copy(agg.at[pl.ds(s * 640, 640)],
                    out_ref.at[pl.ds(c * NP + s * 640, 640)])


@functools.partial(
    pl.kernel,
    out_type=[
        jax.ShapeDtypeStruct((2 * PR, D), jnp.float32),
        jax.ShapeDtypeStruct((NW, 512), jnp.float32),
    ],
    mesh=_mesh,
    compiler_params=pltpu.CompilerParams(needs_layout_passes=False),
    scratch_types=[
        pltpu.VMEM((64, D), jnp.float32),
        pltpu.VMEM((64,), jnp.int32),
        pltpu.VMEM((512,), jnp.float32),
        pltpu.VMEM((24, D), jnp.float32),
        pltpu.VMEM_SHARED((PR, D), jnp.float32),
    ],
)
def _sc_pool(h_ref, b_ref, outp_ref, outc_ref, hbuf, bbuf, cnt, zbuf, pooled):
    c = lax.axis_index("c")
    s = lax.axis_index("s")
    wid = s * 2 + c

    def zc(i, carry):
        cnt[pl.ds(i * 16, 16)] = jnp.zeros((16,), jnp.float32)
        return carry
    lax.fori_loop(0, 32, zc, 0)

    def zrow(r, carry):
        for v in range(8):
            zbuf[r, pl.ds(v * 16, 16)] = jnp.zeros((16,), jnp.float32)
        return carry
    lax.fori_loop(0, 24, zrow, 0)
    pltpu.sync_copy(zbuf, pooled.at[pl.ds(s * 24, 24)])
    plsc.subcore_barrier()

    def chunk(i, carry):
        base = wid * (NP // NW) + i * 64
        pltpu.sync_copy(h_ref.at[pl.ds(base, 64)], hbuf)
        pltpu.sync_copy(b_ref.at[pl.ds(base, 64)], bbuf)
        pltpu.sync_copy(hbuf, pooled.at[bbuf], add=True)

        def grp(g, carry2):
            bv = bbuf[pl.ds(g * 16, 16)]
            plsc.addupdate_scatter(cnt, [bv], jnp.ones((16,), jnp.float32))
            return carry2
        lax.fori_loop(0, 4, grp, 0)
        return carry
    lax.fori_loop(0, NP // NW // 64, chunk, 0)
    plsc.subcore_barrier()
    pltpu.sync_copy(pooled.at[pl.ds(s * 24, 24)],
                    outp_ref.at[pl.ds(c * PR + s * 24, 24)])
    pltpu.sync_copy(cnt, outc_ref.at[wid])


def kernel(x, edge_index, edge_attr, batch, emb, W, W_edge, a, ln_g, ln_b,
           lin1_W, lin1_b, lin2_W, lin2_b):
    n = x.shape[0]
    L = W.shape[0]

    loops = jnp.arange(n, dtype=jnp.int32)
    src = jnp.concatenate([edge_index[0].astype(jnp.int32), loops])
    dst = jnp.concatenate([edge_index[1].astype(jnp.int32), loops])
    pad_e = jnp.full((EP - (E0 + n),), NP - 1, jnp.int32)
    src = jnp.concatenate([src, pad_e])
    dst = jnp.concatenate([dst, pad_e])
    didx = jnp.stack([dst.reshape(EP // C2, C2), src.reshape(EP // C2, C2)],
                     axis=1)

    Wr = W.reshape(L, 2, 2 * PH, D)
    half = jnp.array([0, 1, 0, 1])
    C = jnp.einsum('lhjd,lhj->ldh', Wr[:, half], a)
    C = jnp.concatenate([C, jnp.zeros_like(C)], axis=-1)
    CT = C.transpose(0, 2, 1)
    Wer = W_edge.reshape(L, 2, 2 * PH, D)
    Ce = jnp.einsum('lhjd,lhj->ldh', Wer[:, half[2:]], a[:, 2:])
    Ce_all = Ce.transpose(1, 0, 2).reshape(D, 2 * L)

    es_all = _edge_matmul(edge_attr, Ce_all)
    esT = jnp.zeros((2 * L, EP), jnp.float32).at[:, :E0].set(es_all.T)

    h = jnp.zeros((NP, D), jnp.float32).at[:n].set(emb[x])
    wt_all = W.transpose(0, 2, 1)

    xW, St = _node_matmul(h, wt_all[0], CT[0])
    for l in range(L):
        s_flat = St[:4].reshape(4 * NP)
        es_l = lax.slice(esT, (2 * l, 0), (2 * l + 2, EP)).reshape(2 * EP)
        wE, denoms = _sc_pass1(dst, src, es_l, s_flat)
        inv = 1.0 / (jnp.sum(denoms, axis=0) + 1e-16)
        alphaE = _sc_norm(dst, wE, inv)
        agg2 = _sc_pass2(didx, alphaE, xW)
        agg2 = agg2.reshape(2, NP, D)
        g = ln_g[l].reshape(1, D)
        b = ln_b[l].reshape(1, D)
        if l + 1 < L:
            h, xW, St = _ln_mm(agg2, h, g, b, wt_all[l + 1], CT[l + 1])
        else:
            h = _ln_only(agg2, h, g, b)

    batch_pad = jnp.full((NP,), G, jnp.int32).at[:n].set(batch.astype(jnp.int32))
    p2, cnts = _sc_pool(h, batch_pad)
    hh, out = _mlp(p2, cnts, lin1_W.T, lin1_b.reshape(1, 64),
                   lin2_W.T, lin2_b.reshape(1, 1))
    return out, hh
